# baseline (speedup 1.0000x reference)
import functools

import jax
import jax.numpy as jnp
from jax import lax
from jax.experimental import pallas as pl
from jax.experimental.pallas import tpu as pltpu
from jax.experimental.pallas import tpu_sc as plsc

N = 10000
E = 320000
R = 8
H = 128
FF = 256
L = 2

BN = 1000
_NB = N // BN

_SC_CH = 112
_SC_TSUM = 180
_SC_T0 = 120
_SC_T1 = _SC_TSUM - _SC_T0
_SC_TMAX = max(_SC_T0, _SC_T1)
_E_PAD = 16 * _SC_TSUM * _SC_CH
_NBUF = 3
_NSLOT = 6
_NPAD = 10112

_SQRT1_2 = 0.7071067811865476


def _gelu(x):
    return 0.5 * x * (1.0 + lax.erf(x * _SQRT1_2))


def _ln(x, g, b, eps=1e-5):
    m = jnp.mean(x, axis=-1, keepdims=True)
    v = jnp.mean((x - m) ** 2, axis=-1, keepdims=True)
    return (x - m) / jnp.sqrt(v + eps) * g + b


def _dot(a, b):
    return jnp.dot(a, b, preferred_element_type=jnp.float32)


def _full(shape):
    return pl.BlockSpec(shape, lambda i: tuple(0 for _ in shape))


def _frontend_body(sf, tf, hf, g_sc, b_sc_ln, W_sc, b_sc, W_type, W_hyp,
                   g_fu, b_fu_ln, W_fu, b_fu, out):
    x_sc = _gelu(_dot(_ln(sf[...], g_sc[...], b_sc_ln[...]), W_sc[...]) + b_sc[...])
    x_ty = _gelu(_dot(tf[...], W_type[...]))
    x_hy = _gelu(_dot(hf[...], W_hyp[...]))
    fused = jnp.concatenate([x_sc, x_ty, x_hy], axis=1)
    out[...] = _gelu(_dot(_ln(fused, g_fu[...], b_fu_ln[...]), W_fu[...]) + b_fu[...])


def _tc_frontend(sf, tf, hf, g_sc, b_sc_ln, W_sc, b_sc, W_type, W_hyp,
                 g_fu, b_fu_ln, W_fu, b_fu):
    return pl.pallas_call(
        _frontend_body,
        grid=(_NB,),
        in_specs=[
            pl.BlockSpec((BN, 16), lambda i: (i, 0)),
            pl.BlockSpec((BN, 64), lambda i: (i, 0)),
            pl.BlockSpec((BN, 32), lambda i: (i, 0)),
            _full((1, 16)), _full((1, 16)), _full((16, 16)), _full((1, 16)),
            _full((64, 32)), _full((32, 16)),
            _full((1, 64)), _full((1, 64)), _full((64, H)), _full((1, H)),
        ],
        out_specs=pl.BlockSpec((BN, H), lambda i: (i, 0)),
        out_shape=jax.ShapeDtypeStruct((N, H), jnp.float32),
    )(sf, tf, hf, g_sc, b_sc_ln, W_sc, b_sc, W_type, W_hyp,
      g_fu, b_fu_ln, W_fu, b_fu)


def _proj_body(x, w, out):
    out[0] = _dot(x[...], w[0])


def _tc_proj(x, Wr):
    return pl.pallas_call(
        _proj_body,
        grid=(R, _NB),
        in_specs=[
            pl.BlockSpec((BN, H), lambda r, i: (i, 0)),
            pl.BlockSpec((1, H, H), lambda r, i: (r, 0, 0)),
        ],
        out_specs=pl.BlockSpec((1, BN, H), lambda r, i: (r, i, 0)),
        out_shape=jax.ShapeDtypeStruct((R, N, H), jnp.float32),
    )(x, Wr)


def _sc_segsum(proj_flat, eidx, zeros):
    mesh = plsc.VectorSubcoreMesh(core_axis_name="c", subcore_axis_name="s")

    @functools.partial(
        pl.kernel,
        out_type=jax.ShapeDtypeStruct((2, _NPAD, H), jnp.float32),
        mesh=mesh,
        scratch_types=[
            [pltpu.VMEM((2, _SC_CH), jnp.int32) for _ in range(_NSLOT)],
            [pltpu.VMEM((_SC_CH, H), jnp.float32) for _ in range(_NBUF)],
            pltpu.VMEM_SHARED((_NPAD, H), jnp.float32),
            [pltpu.SemaphoreType.DMA for _ in range(_NSLOT)],
            [pltpu.SemaphoreType.DMA for _ in range(_NBUF)],
            [pltpu.SemaphoreType.DMA for _ in range(_NBUF)],
        ],
    )
    def k(proj_hbm, eidx_hbm, zeros_hbm, out_hbm,
          slots, bufs, agg_sh, sis, gsem, ssem):
        c = lax.axis_index("c")
        s = lax.axis_index("s")
        wid = c * 16 + s
        my_eidx = eidx_hbm.at[wid]
        zr = _NPAD // 16
        pltpu.sync_copy(zeros_hbm.at[pl.ds(s * zr, zr)],
                        agg_sh.at[pl.ds(s * zr, zr)])

        for j in range(3):
            pltpu.async_copy(my_eidx.at[j], slots[j], sis[j])
        for j in range(2):
            pltpu.make_async_copy(my_eidx.at[0], slots[j], sis[j]).wait()
            pltpu.async_copy(proj_hbm.at[slots[j].at[0]], bufs[j], gsem[j])

        plsc.subcore_barrier()

        def chunk_work(g, j, T):
            m = g + j
            b = j % _NBUF
            sl = slots[j % _NSLOT]
            pltpu.make_async_copy(proj_hbm.at[sl.at[0]], bufs[b], gsem[b]).wait()
            pltpu.async_copy(bufs[b], agg_sh.at[sl.at[1]], ssem[b], add=True)

            jf = (j + 3) % _NSLOT

            @pl.when(m + 3 < T)
            def _():
                pltpu.async_copy(my_eidx.at[m + 3], slots[jf], sis[jf])

            @pl.when(m >= 1)
            def _():
                b1 = (j + _NBUF - 1) % _NBUF
                pltpu.make_async_copy(bufs[b1], agg_sh.at[sl.at[1]],
                                      ssem[b1]).wait()

            jn = (j + 2) % _NSLOT

            @pl.when(m + 2 < T)
            def _():
                b2 = (j + 2) % _NBUF
                pltpu.make_async_copy(my_eidx.at[0], slots[jn], sis[jn]).wait()
                pltpu.async_copy(proj_hbm.at[slots[jn].at[0]], bufs[b2],
                                 gsem[b2])

        def run(T):
            @pl.loop(0, T, step=_NSLOT)
            def _(g):
                for j in range(_NSLOT):
                    chunk_work(g, j, T)
            b_last = (T - 1) % _NBUF
            pltpu.make_async_copy(bufs[b_last], agg_sh.at[slots[0].at[1]],
                                  ssem[b_last]).wait()

        @pl.when(c == 0)
        def _():
            run(_SC_T0)

        @pl.when(c == 1)
        def _():
            run(_SC_T1)

        plsc.subcore_barrier()
        pltpu.sync_copy(agg_sh.at[pl.ds(s * zr, zr)],
                        out_hbm.at[c].at[pl.ds(s * zr, zr)])

    return k(proj_flat, eidx, zeros)


def _post_body(p, x, wl, bc, gp, bp, gf, bf, w1, b1, w2, b2, out):
    xv = x[...]
    h = p[0] + p[1] + _dot(xv, wl[...]) + bc[...]
    h = _gelu(_ln(h, gp[...], bp[...]))
    x1 = xv + h
    y = _ln(x1, gf[...], bf[...])
    y = _dot(_gelu(_dot(y, w1[...]) + b1[...]), w2[...]) + b2[...]
    out[...] = x1 + y


def _tc_post(part, x, wl, bc, gp, bp, gf, bf, w1, b1, w2, b2):
    return pl.pallas_call(
        _post_body,
        grid=(_NB,),
        in_specs=[
            pl.BlockSpec((2, BN, H), lambda i: (0, i, 0)),
            pl.BlockSpec((BN, H), lambda i: (i, 0)),
            _full((H, H)), _full((1, H)),
            _full((1, H)), _full((1, H)), _full((1, H)), _full((1, H)),
            _full((H, FF)), _full((1, FF)), _full((FF, H)), _full((1, H)),
        ],
        out_specs=pl.BlockSpec((BN, H), lambda i: (i, 0)),
        out_shape=jax.ShapeDtypeStruct((N, H), jnp.float32),
    )(part, x, wl, bc, gp, bp, gf, bf, w1, b1, w2, b2)


def _heads_body(x, gk, bk, wk1, bk1, wk2, bk2, gp, bp, wp1, bp1, wp2, bp2,
                keep, pri):
    xv = x[...]
    hk = _gelu(_dot(_ln(xv, gk[...], bk[...]), wk1[...]) + bk1[...])
    keep[...] = jnp.sum(hk * wk2[...], axis=1, keepdims=True) + bk2[...]
    hp = _gelu(_dot(_ln(xv, gp[...], bp[...]), wp1[...]) + bp1[...])
    pri[...] = jnp.sum(hp * wp2[...], axis=1, keepdims=True) + bp2[...]


def _tc_heads(x, gk, bk, wk1, bk1, wk2, bk2, gp, bp, wp1, bp1, wp2, bp2):
    return pl.pallas_call(
        _heads_body,
        grid=(_NB,),
        in_specs=[
            pl.BlockSpec((BN, H), lambda i: (i, 0)),
            _full((1, H)), _full((1, H)), _full((H, H)), _full((1, H)),
            _full((1, H)), _full((1, 1)),
            _full((1, H)), _full((1, H)), _full((H, H)), _full((1, H)),
            _full((1, H)), _full((1, 1)),
        ],
        out_specs=[
            pl.BlockSpec((BN, 1), lambda i: (i, 0)),
            pl.BlockSpec((BN, 1), lambda i: (i, 0)),
        ],
        out_shape=[
            jax.ShapeDtypeStruct((N, 1), jnp.float32),
            jax.ShapeDtypeStruct((N, 1), jnp.float32),
        ],
    )(x, gk, bk, wk1, bk1, wk2, bk2, gp, bp, wp1, bp1, wp2, bp2)


def _row(v):
    return v.reshape(1, -1)


def kernel(scalar_feat, type_feat, hyp_feat, edge_index, rel_type, ln_sc_g, ln_sc_b, W_sc, b_sc, W_type, W_hyp, ln_fu_g, ln_fu_b, W_fu, b_fu, Wrel, Wloop, b_conv, ln_pre_g, ln_pre_b, ln_ff_g, ln_ff_b, W_ff1, b_ff1, W_ff2, b_ff2, ln_k_g, ln_k_b, Wk1, bk1, Wk2, bk2, ln_p_g, ln_p_b, Wp1, bp1, Wp2, bp2):
    src = edge_index[0]
    dst = edge_index[1]
    gidx = rel_type * N + src
    pad = _E_PAD - E
    gidx_p = jnp.concatenate([gidx, jnp.zeros((pad,), jnp.int32)])
    dst_p = jnp.concatenate([dst, jnp.full((pad,), N, jnp.int32)])
    e0 = 16 * _SC_T0 * _SC_CH

    def _worker_tables(flat, t_cnt, lo, hi):
        w = flat[lo:hi].reshape(16, t_cnt, _SC_CH)
        if t_cnt < _SC_TMAX:
            w = jnp.pad(w, ((0, 0), (0, _SC_TMAX - t_cnt), (0, 0)))
        return w

    eidx = jnp.stack(
        [jnp.concatenate([_worker_tables(a, _SC_T0, 0, e0),
                          _worker_tables(a, _SC_T1, e0, _E_PAD)], axis=0)
         for a in (gidx_p, dst_p)], axis=2)
    zeros = jnp.zeros((_NPAD, H), jnp.float32)

    x = _tc_frontend(scalar_feat, type_feat, hyp_feat,
                     _row(ln_sc_g), _row(ln_sc_b), W_sc, _row(b_sc),
                     W_type, W_hyp,
                     _row(ln_fu_g), _row(ln_fu_b), W_fu, _row(b_fu))

    for i in range(L):
        proj = _tc_proj(x, Wrel[i])
        part = _sc_segsum(proj.reshape(R * N, H), eidx, zeros)
        x = _tc_post(part, x, Wloop[i], _row(b_conv[i]),
                     _row(ln_pre_g[i]), _row(ln_pre_b[i]),
                     _row(ln_ff_g[i]), _row(ln_ff_b[i]),
                     W_ff1[i], _row(b_ff1[i]), W_ff2[i], _row(b_ff2[i]))

    keep, pri = _tc_heads(x, _row(ln_k_g), _row(ln_k_b), Wk1, _row(bk1),
                          _row(Wk2), bk2.reshape(1, 1),
                          _row(ln_p_g), _row(ln_p_b),
                          Wp1, _row(bp1), _row(Wp2), bp2.reshape(1, 1))
    return jnp.stack([keep[:, 0], pri[:, 0]], axis=0)


# speedup vs baseline: 1.1658x; 1.1658x over previous
import functools

import jax
import jax.numpy as jnp
from jax import lax
from jax.experimental import pallas as pl
from jax.experimental.pallas import tpu as pltpu
from jax.experimental.pallas import tpu_sc as plsc

N = 10000
E = 320000
R = 8
H = 128
FF = 256
L = 2

BN = 1000
_NB = N // BN

_SC_CH = 112
_SC_TSUM = 180
_SC_T0 = 138
_SC_T1 = _SC_TSUM - _SC_T0
_SC_TMAX = max(_SC_T0, _SC_T1)
_E_PAD = 16 * _SC_TSUM * _SC_CH
_NBUF = 3
_NSLOT = 6
_NPAD = 10112

_SQRT1_2 = 0.7071067811865476


def _gelu(x):
    return 0.5 * x * (1.0 + lax.erf(x * _SQRT1_2))


def _ln(x, g, b, eps=1e-5):
    m = jnp.mean(x, axis=-1, keepdims=True)
    v = jnp.mean((x - m) ** 2, axis=-1, keepdims=True)
    return (x - m) / jnp.sqrt(v + eps) * g + b


def _dot(a, b):
    return jnp.dot(a, b, preferred_element_type=jnp.float32)


def _full(shape):
    return pl.BlockSpec(shape, lambda i: tuple(0 for _ in shape))


def _frontend_body(sf, tf, hf, g_sc, b_sc_ln, W_sc, b_sc, W_type, W_hyp,
                   g_fu, b_fu_ln, W_fu, b_fu, out):
    x_sc = _gelu(_dot(_ln(sf[...], g_sc[...], b_sc_ln[...]), W_sc[...]) + b_sc[...])
    x_ty = _gelu(_dot(tf[...], W_type[...]))
    x_hy = _gelu(_dot(hf[...], W_hyp[...]))
    fused = jnp.concatenate([x_sc, x_ty, x_hy], axis=1)
    out[...] = _gelu(_dot(_ln(fused, g_fu[...], b_fu_ln[...]), W_fu[...]) + b_fu[...])


def _tc_frontend(sf, tf, hf, g_sc, b_sc_ln, W_sc, b_sc, W_type, W_hyp,
                 g_fu, b_fu_ln, W_fu, b_fu):
    return pl.pallas_call(
        _frontend_body,
        grid=(_NB,),
        in_specs=[
            pl.BlockSpec((BN, 16), lambda i: (i, 0)),
            pl.BlockSpec((BN, 64), lambda i: (i, 0)),
            pl.BlockSpec((BN, 32), lambda i: (i, 0)),
            _full((1, 16)), _full((1, 16)), _full((16, 16)), _full((1, 16)),
            _full((64, 32)), _full((32, 16)),
            _full((1, 64)), _full((1, 64)), _full((64, H)), _full((1, H)),
        ],
        out_specs=pl.BlockSpec((BN, H), lambda i: (i, 0)),
        out_shape=jax.ShapeDtypeStruct((N, H), jnp.float32),
    )(sf, tf, hf, g_sc, b_sc_ln, W_sc, b_sc, W_type, W_hyp,
      g_fu, b_fu_ln, W_fu, b_fu)


def _proj_body(x, w, out):
    xb = x[...].astype(jnp.bfloat16)
    for r in range(R):
        out[r] = jnp.dot(xb, w[r].astype(jnp.bfloat16),
                         preferred_element_type=jnp.float32)


def _tc_proj(x, Wr):
    return pl.pallas_call(
        _proj_body,
        grid=(_NB,),
        in_specs=[
            pl.BlockSpec((BN, H), lambda i: (i, 0)),
            pl.BlockSpec((R, H, H), lambda i: (0, 0, 0)),
        ],
        out_specs=pl.BlockSpec((R, BN, H), lambda i: (0, i, 0)),
        out_shape=jax.ShapeDtypeStruct((R, N, H), jnp.float32),
    )(x, Wr)


def _sc_segsum(proj_flat, eidx, zeros):
    mesh = plsc.VectorSubcoreMesh(core_axis_name="c", subcore_axis_name="s")

    @functools.partial(
        pl.kernel,
        out_type=jax.ShapeDtypeStruct((2, _NPAD, H), jnp.float32),
        mesh=mesh,
        scratch_types=[
            [pltpu.VMEM((2, _SC_CH), jnp.int32) for _ in range(_NSLOT)],
            [pltpu.VMEM((_SC_CH, H), jnp.float32) for _ in range(_NBUF)],
            pltpu.VMEM_SHARED((_NPAD, H), jnp.float32),
            [pltpu.SemaphoreType.DMA for _ in range(_NSLOT)],
            [pltpu.SemaphoreType.DMA for _ in range(_NBUF)],
            [pltpu.SemaphoreType.DMA for _ in range(_NBUF)],
        ],
    )
    def k(proj_hbm, eidx_hbm, zeros_hbm, out_hbm,
          slots, bufs, agg_sh, sis, gsem, ssem):
        c = lax.axis_index("c")
        s = lax.axis_index("s")
        wid = c * 16 + s
        my_eidx = eidx_hbm.at[wid]
        zr = _NPAD // 16
        pltpu.sync_copy(zeros_hbm.at[pl.ds(s * zr, zr)],
                        agg_sh.at[pl.ds(s * zr, zr)])

        for j in range(3):
            pltpu.async_copy(my_eidx.at[j], slots[j], sis[j])
        for j in range(2):
            pltpu.make_async_copy(my_eidx.at[0], slots[j], sis[j]).wait()
            pltpu.async_copy(proj_hbm.at[slots[j].at[0]], bufs[j], gsem[j])

        plsc.subcore_barrier()

        def chunk_work(g, j, T):
            m = g + j
            b = j % _NBUF
            sl = slots[j % _NSLOT]
            pltpu.make_async_copy(proj_hbm.at[sl.at[0]], bufs[b], gsem[b]).wait()
            pltpu.async_copy(bufs[b], agg_sh.at[sl.at[1]], ssem[b], add=True)

            jf = (j + 3) % _NSLOT

            @pl.when(m + 3 < T)
            def _():
                pltpu.async_copy(my_eidx.at[m + 3], slots[jf], sis[jf])

            @pl.when(m >= 1)
            def _():
                b1 = (j + _NBUF - 1) % _NBUF
                pltpu.make_async_copy(bufs[b1], agg_sh.at[sl.at[1]],
                                      ssem[b1]).wait()

            jn = (j + 2) % _NSLOT

            @pl.when(m + 2 < T)
            def _():
                b2 = (j + 2) % _NBUF
                pltpu.make_async_copy(my_eidx.at[0], slots[jn], sis[jn]).wait()
                pltpu.async_copy(proj_hbm.at[slots[jn].at[0]], bufs[b2],
                                 gsem[b2])

        def run(T):
            @pl.loop(0, T, step=_NSLOT)
            def _(g):
                for j in range(_NSLOT):
                    chunk_work(g, j, T)
            b_last = (T - 1) % _NBUF
            pltpu.make_async_copy(bufs[b_last], agg_sh.at[slots[0].at[1]],
                                  ssem[b_last]).wait()

        @pl.when(c == 0)
        def _():
            run(_SC_T0)

        @pl.when(c == 1)
        def _():
            run(_SC_T1)

        plsc.subcore_barrier()
        pltpu.sync_copy(agg_sh.at[pl.ds(s * zr, zr)],
                        out_hbm.at[c].at[pl.ds(s * zr, zr)])

    return k(proj_flat, eidx, zeros)


def _post_body(p, x, wl, bc, gp, bp, gf, bf, w1, b1, w2, b2, out):
    xv = x[...]
    h = p[0] + p[1] + _dot(xv, wl[...]) + bc[...]
    h = _gelu(_ln(h, gp[...], bp[...]))
    x1 = xv + h
    y = _ln(x1, gf[...], bf[...])
    y = _dot(_gelu(_dot(y, w1[...]) + b1[...]), w2[...]) + b2[...]
    out[...] = x1 + y


def _tc_post(part, x, wl, bc, gp, bp, gf, bf, w1, b1, w2, b2):
    return pl.pallas_call(
        _post_body,
        grid=(_NB,),
        in_specs=[
            pl.BlockSpec((2, BN, H), lambda i: (0, i, 0)),
            pl.BlockSpec((BN, H), lambda i: (i, 0)),
            _full((H, H)), _full((1, H)),
            _full((1, H)), _full((1, H)), _full((1, H)), _full((1, H)),
            _full((H, FF)), _full((1, FF)), _full((FF, H)), _full((1, H)),
        ],
        out_specs=pl.BlockSpec((BN, H), lambda i: (i, 0)),
        out_shape=jax.ShapeDtypeStruct((N, H), jnp.float32),
    )(part, x, wl, bc, gp, bp, gf, bf, w1, b1, w2, b2)


def _heads_body(x, gk, bk, wk1, bk1, wk2, bk2, gp, bp, wp1, bp1, wp2, bp2,
                keep, pri):
    xv = x[...]
    hk = _gelu(_dot(_ln(xv, gk[...], bk[...]), wk1[...]) + bk1[...])
    keep[...] = jnp.sum(hk * wk2[...], axis=1, keepdims=True) + bk2[...]
    hp = _gelu(_dot(_ln(xv, gp[...], bp[...]), wp1[...]) + bp1[...])
    pri[...] = jnp.sum(hp * wp2[...], axis=1, keepdims=True) + bp2[...]


def _tc_heads(x, gk, bk, wk1, bk1, wk2, bk2, gp, bp, wp1, bp1, wp2, bp2):
    return pl.pallas_call(
        _heads_body,
        grid=(_NB,),
        in_specs=[
            pl.BlockSpec((BN, H), lambda i: (i, 0)),
            _full((1, H)), _full((1, H)), _full((H, H)), _full((1, H)),
            _full((1, H)), _full((1, 1)),
            _full((1, H)), _full((1, H)), _full((H, H)), _full((1, H)),
            _full((1, H)), _full((1, 1)),
        ],
        out_specs=[
            pl.BlockSpec((BN, 1), lambda i: (i, 0)),
            pl.BlockSpec((BN, 1), lambda i: (i, 0)),
        ],
        out_shape=[
            jax.ShapeDtypeStruct((N, 1), jnp.float32),
            jax.ShapeDtypeStruct((N, 1), jnp.float32),
        ],
    )(x, gk, bk, wk1, bk1, wk2, bk2, gp, bp, wp1, bp1, wp2, bp2)


def _row(v):
    return v.reshape(1, -1)


def kernel(scalar_feat, type_feat, hyp_feat, edge_index, rel_type, ln_sc_g, ln_sc_b, W_sc, b_sc, W_type, W_hyp, ln_fu_g, ln_fu_b, W_fu, b_fu, Wrel, Wloop, b_conv, ln_pre_g, ln_pre_b, ln_ff_g, ln_ff_b, W_ff1, b_ff1, W_ff2, b_ff2, ln_k_g, ln_k_b, Wk1, bk1, Wk2, bk2, ln_p_g, ln_p_b, Wp1, bp1, Wp2, bp2):
    src = edge_index[0]
    dst = edge_index[1]
    gidx = rel_type * N + src
    pad = _E_PAD - E
    gidx_p = jnp.concatenate([gidx, jnp.zeros((pad,), jnp.int32)])
    dst_p = jnp.concatenate([dst, jnp.full((pad,), N, jnp.int32)])
    e0 = 16 * _SC_T0 * _SC_CH

    def _worker_tables(flat, t_cnt, lo, hi):
        w = flat[lo:hi].reshape(16, t_cnt, _SC_CH)
        if t_cnt < _SC_TMAX:
            w = jnp.pad(w, ((0, 0), (0, _SC_TMAX - t_cnt), (0, 0)))
        return w

    eidx = jnp.stack(
        [jnp.concatenate([_worker_tables(a, _SC_T0, 0, e0),
                          _worker_tables(a, _SC_T1, e0, _E_PAD)], axis=0)
         for a in (gidx_p, dst_p)], axis=2)
    zeros = jnp.zeros((_NPAD, H), jnp.float32)

    x = _tc_frontend(scalar_feat, type_feat, hyp_feat,
                     _row(ln_sc_g), _row(ln_sc_b), W_sc, _row(b_sc),
                     W_type, W_hyp,
                     _row(ln_fu_g), _row(ln_fu_b), W_fu, _row(b_fu))

    for i in range(L):
        proj = _tc_proj(x, Wrel[i])
        part = _sc_segsum(proj.reshape(R * N, H), eidx, zeros)
        x = _tc_post(part, x, Wloop[i], _row(b_conv[i]),
                     _row(ln_pre_g[i]), _row(ln_pre_b[i]),
                     _row(ln_ff_g[i]), _row(ln_ff_b[i]),
                     W_ff1[i], _row(b_ff1[i]), W_ff2[i], _row(b_ff2[i]))

    keep, pri = _tc_heads(x, _row(ln_k_g), _row(ln_k_b), Wk1, _row(bk1),
                          _row(Wk2), bk2.reshape(1, 1),
                          _row(ln_p_g), _row(ln_p_b),
                          Wp1, _row(bp1), _row(Wp2), bp2.reshape(1, 1))
    return jnp.stack([keep[:, 0], pri[:, 0]], axis=0)


# speedup vs baseline: 1.2051x; 1.0337x over previous
import functools

import jax
import jax.numpy as jnp
from jax import lax
from jax.experimental import pallas as pl
from jax.experimental.pallas import tpu as pltpu
from jax.experimental.pallas import tpu_sc as plsc

N = 10000
E = 320000
R = 8
H = 128
FF = 256
L = 2

BN = 1000
_NB = N // BN

_SC_CH = 112
_SC_TSUM = 180
_SC_T0 = 144
_SC_T1 = _SC_TSUM - _SC_T0
_SC_TMAX = max(_SC_T0, _SC_T1)
_E_PAD = 16 * _SC_TSUM * _SC_CH
_NBUF = 3
_NSLOT = 6
_NPAD = 10112

_SQRT1_2 = 0.7071067811865476


def _gelu(x):
    return 0.5 * x * (1.0 + lax.erf(x * _SQRT1_2))


def _ln(x, g, b, eps=1e-5):
    m = jnp.mean(x, axis=-1, keepdims=True)
    v = jnp.mean((x - m) ** 2, axis=-1, keepdims=True)
    return (x - m) / jnp.sqrt(v + eps) * g + b


def _dot(a, b):
    return jnp.dot(a, b, preferred_element_type=jnp.float32)


def _full(shape):
    return pl.BlockSpec(shape, lambda i: tuple(0 for _ in shape))


def _frontend_body(sf, tf, hf, g_sc, b_sc_ln, Wcat, bcat,
                   g_fu, b_fu_ln, W_fu, b_fu, out):
    a = jnp.concatenate(
        [_ln(sf[...], g_sc[...], b_sc_ln[...]), tf[...], hf[...]], axis=1)
    fused = _gelu(_dot(a, Wcat[...]) + bcat[...])
    out[...] = _gelu(_dot(_ln(fused, g_fu[...], b_fu_ln[...]), W_fu[...]) + b_fu[...])


def _tc_frontend(sf, tf, hf, g_sc, b_sc_ln, Wcat, bcat,
                 g_fu, b_fu_ln, W_fu, b_fu):
    return pl.pallas_call(
        _frontend_body,
        grid=(_NB,),
        in_specs=[
            pl.BlockSpec((BN, 16), lambda i: (i, 0)),
            pl.BlockSpec((BN, 64), lambda i: (i, 0)),
            pl.BlockSpec((BN, 32), lambda i: (i, 0)),
            _full((1, 16)), _full((1, 16)), _full((112, 64)), _full((1, 64)),
            _full((1, 64)), _full((1, 64)), _full((64, H)), _full((1, H)),
        ],
        out_specs=pl.BlockSpec((BN, H), lambda i: (i, 0)),
        out_shape=jax.ShapeDtypeStruct((N, H), jnp.float32),
    )(sf, tf, hf, g_sc, b_sc_ln, Wcat, bcat,
      g_fu, b_fu_ln, W_fu, b_fu)


def _proj_body(x, w, out):
    xb = x[...].astype(jnp.bfloat16)
    for r in range(R):
        out[r] = jnp.dot(xb, w[r].astype(jnp.bfloat16),
                         preferred_element_type=jnp.float32)


def _tc_proj(x, Wr):
    return pl.pallas_call(
        _proj_body,
        grid=(_NB,),
        in_specs=[
            pl.BlockSpec((BN, H), lambda i: (i, 0)),
            pl.BlockSpec((R, H, H), lambda i: (0, 0, 0)),
        ],
        out_specs=pl.BlockSpec((R, BN, H), lambda i: (0, i, 0)),
        out_shape=jax.ShapeDtypeStruct((R, N, H), jnp.float32),
    )(x, Wr)


def _sc_segsum(proj_flat, eidx, zeros):
    mesh = plsc.VectorSubcoreMesh(core_axis_name="c", subcore_axis_name="s")

    @functools.partial(
        pl.kernel,
        out_type=jax.ShapeDtypeStruct((2, _NPAD, H), jnp.float32),
        mesh=mesh,
        scratch_types=[
            [pltpu.VMEM((2, _SC_CH), jnp.int32) for _ in range(_NSLOT)],
            [pltpu.VMEM((_SC_CH, H), jnp.float32) for _ in range(_NBUF)],
            pltpu.VMEM_SHARED((_NPAD, H), jnp.float32),
            [pltpu.SemaphoreType.DMA for _ in range(_NSLOT)],
            [pltpu.SemaphoreType.DMA for _ in range(_NBUF)],
            [pltpu.SemaphoreType.DMA for _ in range(_NBUF)],
        ],
    )
    def k(proj_hbm, eidx_hbm, zeros_hbm, out_hbm,
          slots, bufs, agg_sh, sis, gsem, ssem):
        c = lax.axis_index("c")
        s = lax.axis_index("s")
        wid = c * 16 + s
        my_eidx = eidx_hbm.at[wid]
        zr = _NPAD // 16
        pltpu.sync_copy(zeros_hbm.at[pl.ds(s * zr, zr)],
                        agg_sh.at[pl.ds(s * zr, zr)])

        for j in range(3):
            pltpu.async_copy(my_eidx.at[j], slots[j], sis[j])
        for j in range(2):
            pltpu.make_async_copy(my_eidx.at[0], slots[j], sis[j]).wait()
            pltpu.async_copy(proj_hbm.at[slots[j].at[0]], bufs[j], gsem[j])

        plsc.subcore_barrier()

        def chunk_work(g, j, T):
            m = g + j
            b = j % _NBUF
            sl = slots[j % _NSLOT]
            pltpu.make_async_copy(proj_hbm.at[sl.at[0]], bufs[b], gsem[b]).wait()
            pltpu.async_copy(bufs[b], agg_sh.at[sl.at[1]], ssem[b], add=True)

            jf = (j + 3) % _NSLOT

            @pl.when(m + 3 < T)
            def _():
                pltpu.async_copy(my_eidx.at[m + 3], slots[jf], sis[jf])

            @pl.when(m >= 1)
            def _():
                b1 = (j + _NBUF - 1) % _NBUF
                pltpu.make_async_copy(bufs[b1], agg_sh.at[sl.at[1]],
                                      ssem[b1]).wait()

            jn = (j + 2) % _NSLOT

            @pl.when(m + 2 < T)
            def _():
                b2 = (j + 2) % _NBUF
                pltpu.make_async_copy(my_eidx.at[0], slots[jn], sis[jn]).wait()
                pltpu.async_copy(proj_hbm.at[slots[jn].at[0]], bufs[b2],
                                 gsem[b2])

        def run(T):
            @pl.loop(0, T, step=_NSLOT)
            def _(g):
                for j in range(_NSLOT):
                    chunk_work(g, j, T)
            b_last = (T - 1) % _NBUF
            pltpu.make_async_copy(bufs[b_last], agg_sh.at[slots[0].at[1]],
                                  ssem[b_last]).wait()

        @pl.when(c == 0)
        def _():
            run(_SC_T0)

        @pl.when(c == 1)
        def _():
            run(_SC_T1)

        plsc.subcore_barrier()
        pltpu.sync_copy(agg_sh.at[pl.ds(s * zr, zr)],
                        out_hbm.at[c].at[pl.ds(s * zr, zr)])

    return k(proj_flat, eidx, zeros)


def _post_body(p, x, wl, bc, gp, bp, gf, bf, w1, b1, w2, b2, out):
    xv = x[...]
    h = p[0] + p[1] + _dot(xv, wl[...]) + bc[...]
    h = _gelu(_ln(h, gp[...], bp[...]))
    x1 = xv + h
    y = _ln(x1, gf[...], bf[...])
    y = _dot(_gelu(_dot(y, w1[...]) + b1[...]), w2[...]) + b2[...]
    out[...] = x1 + y


def _tc_post(part, x, wl, bc, gp, bp, gf, bf, w1, b1, w2, b2):
    return pl.pallas_call(
        _post_body,
        grid=(_NB,),
        in_specs=[
            pl.BlockSpec((2, BN, H), lambda i: (0, i, 0)),
            pl.BlockSpec((BN, H), lambda i: (i, 0)),
            _full((H, H)), _full((1, H)),
            _full((1, H)), _full((1, H)), _full((1, H)), _full((1, H)),
            _full((H, FF)), _full((1, FF)), _full((FF, H)), _full((1, H)),
        ],
        out_specs=pl.BlockSpec((BN, H), lambda i: (i, 0)),
        out_shape=jax.ShapeDtypeStruct((N, H), jnp.float32),
    )(part, x, wl, bc, gp, bp, gf, bf, w1, b1, w2, b2)


def _heads_body(x, gk, bk, wk1, bk1, wk2, bk2, gp, bp, wp1, bp1, wp2, bp2,
                keep, pri):
    xv = x[...]
    hk = _gelu(_dot(_ln(xv, gk[...], bk[...]), wk1[...]) + bk1[...])
    keep[...] = jnp.sum(hk * wk2[...], axis=1, keepdims=True) + bk2[...]
    hp = _gelu(_dot(_ln(xv, gp[...], bp[...]), wp1[...]) + bp1[...])
    pri[...] = jnp.sum(hp * wp2[...], axis=1, keepdims=True) + bp2[...]


def _tc_heads(x, gk, bk, wk1, bk1, wk2, bk2, gp, bp, wp1, bp1, wp2, bp2):
    return pl.pallas_call(
        _heads_body,
        grid=(_NB,),
        in_specs=[
            pl.BlockSpec((BN, H), lambda i: (i, 0)),
            _full((1, H)), _full((1, H)), _full((H, H)), _full((1, H)),
            _full((1, H)), _full((1, 1)),
            _full((1, H)), _full((1, H)), _full((H, H)), _full((1, H)),
            _full((1, H)), _full((1, 1)),
        ],
        out_specs=[
            pl.BlockSpec((BN, 1), lambda i: (i, 0)),
            pl.BlockSpec((BN, 1), lambda i: (i, 0)),
        ],
        out_shape=[
            jax.ShapeDtypeStruct((N, 1), jnp.float32),
            jax.ShapeDtypeStruct((N, 1), jnp.float32),
        ],
    )(x, gk, bk, wk1, bk1, wk2, bk2, gp, bp, wp1, bp1, wp2, bp2)


def _row(v):
    return v.reshape(1, -1)


def kernel(scalar_feat, type_feat, hyp_feat, edge_index, rel_type, ln_sc_g, ln_sc_b, W_sc, b_sc, W_type, W_hyp, ln_fu_g, ln_fu_b, W_fu, b_fu, Wrel, Wloop, b_conv, ln_pre_g, ln_pre_b, ln_ff_g, ln_ff_b, W_ff1, b_ff1, W_ff2, b_ff2, ln_k_g, ln_k_b, Wk1, bk1, Wk2, bk2, ln_p_g, ln_p_b, Wp1, bp1, Wp2, bp2):
    src = edge_index[0]
    dst = edge_index[1]
    gidx = rel_type * N + src
    pad = _E_PAD - E
    gidx_p = jnp.concatenate([gidx, jnp.zeros((pad,), jnp.int32)])
    dst_p = jnp.concatenate([dst, jnp.full((pad,), N, jnp.int32)])
    e0 = 16 * _SC_T0 * _SC_CH

    def _worker_tables(flat, t_cnt, lo, hi):
        w = flat[lo:hi].reshape(16, t_cnt, _SC_CH)
        if t_cnt < _SC_TMAX:
            w = jnp.pad(w, ((0, 0), (0, _SC_TMAX - t_cnt), (0, 0)))
        return w

    eidx = jnp.stack(
        [jnp.concatenate([_worker_tables(a, _SC_T0, 0, e0),
                          _worker_tables(a, _SC_T1, e0, _E_PAD)], axis=0)
         for a in (gidx_p, dst_p)], axis=2)
    zeros = jnp.zeros((_NPAD, H), jnp.float32)

    Wcat = jnp.zeros((112, 64), jnp.float32)
    Wcat = Wcat.at[:16, :16].set(W_sc)
    Wcat = Wcat.at[16:80, 16:48].set(W_type)
    Wcat = Wcat.at[80:, 48:].set(W_hyp)
    bcat = jnp.concatenate([b_sc, jnp.zeros((48,), jnp.float32)])

    x = _tc_frontend(scalar_feat, type_feat, hyp_feat,
                     _row(ln_sc_g), _row(ln_sc_b), Wcat, _row(bcat),
                     _row(ln_fu_g), _row(ln_fu_b), W_fu, _row(b_fu))

    for i in range(L):
        proj = _tc_proj(x, Wrel[i])
        part = _sc_segsum(proj.reshape(R * N, H), eidx, zeros)
        x = _tc_post(part, x, Wloop[i], _row(b_conv[i]),
                     _row(ln_pre_g[i]), _row(ln_pre_b[i]),
                     _row(ln_ff_g[i]), _row(ln_ff_b[i]),
                     W_ff1[i], _row(b_ff1[i]), W_ff2[i], _row(b_ff2[i]))

    keep, pri = _tc_heads(x, _row(ln_k_g), _row(ln_k_b), Wk1, _row(bk1),
                          _row(Wk2), bk2.reshape(1, 1),
                          _row(ln_p_g), _row(ln_p_b),
                          Wp1, _row(bp1), _row(Wp2), bp2.reshape(1, 1))
    return jnp.stack([keep[:, 0], pri[:, 0]], axis=0)


# speedup vs baseline: 1.2300x; 1.0206x over previous
import functools

import jax
import jax.numpy as jnp
from jax import lax
from jax.experimental import pallas as pl
from jax.experimental.pallas import tpu as pltpu
from jax.experimental.pallas import tpu_sc as plsc

N = 10000
E = 320000
R = 8
H = 128
FF = 256
L = 2

BN = 1000
_NB = N // BN

_SC_CH = 112
_SC_TSUM = 180
_SC_T0 = 144
_SC_T1 = _SC_TSUM - _SC_T0
_SC_TMAX = max(_SC_T0, _SC_T1)
_E_PAD = 16 * _SC_TSUM * _SC_CH
_NBUF = 3
_NSLOT = 6
_NPAD = 10112

_SQRT1_2 = 0.7071067811865476


def _gelu(x):
    return 0.5 * x * (1.0 + lax.erf(x * _SQRT1_2))


def _ln(x, g, b, eps=1e-5):
    m = jnp.mean(x, axis=-1, keepdims=True)
    v = jnp.mean((x - m) ** 2, axis=-1, keepdims=True)
    return (x - m) / jnp.sqrt(v + eps) * g + b


def _dot(a, b):
    return jnp.dot(a, b, preferred_element_type=jnp.float32)


def _full(shape):
    return pl.BlockSpec(shape, lambda i: tuple(0 for _ in shape))


def _frontend_body(sf, tf, hf, g_sc, b_sc_ln, Wcat, bcat,
                   g_fu, b_fu_ln, W_fu, b_fu, out):
    a = jnp.concatenate(
        [_ln(sf[...], g_sc[...], b_sc_ln[...]), tf[...], hf[...]], axis=1)
    fused = _gelu(_dot(a, Wcat[...]) + bcat[...])
    out[...] = _gelu(_dot(_ln(fused, g_fu[...], b_fu_ln[...]), W_fu[...]) + b_fu[...])


def _tc_frontend(sf, tf, hf, g_sc, b_sc_ln, Wcat, bcat,
                 g_fu, b_fu_ln, W_fu, b_fu):
    return pl.pallas_call(
        _frontend_body,
        grid=(_NB,),
        in_specs=[
            pl.BlockSpec((BN, 16), lambda i: (i, 0)),
            pl.BlockSpec((BN, 64), lambda i: (i, 0)),
            pl.BlockSpec((BN, 32), lambda i: (i, 0)),
            _full((1, 16)), _full((1, 16)), _full((112, 64)), _full((1, 64)),
            _full((1, 64)), _full((1, 64)), _full((64, H)), _full((1, H)),
        ],
        out_specs=pl.BlockSpec((BN, H), lambda i: (i, 0)),
        out_shape=jax.ShapeDtypeStruct((N, H), jnp.float32),
    )(sf, tf, hf, g_sc, b_sc_ln, Wcat, bcat,
      g_fu, b_fu_ln, W_fu, b_fu)


def _proj_body(x, w, out):
    xb = x[...].astype(jnp.bfloat16)
    for r in range(R):
        out[r] = jnp.dot(xb, w[r].astype(jnp.bfloat16),
                         preferred_element_type=jnp.float32)


def _tc_proj(x, Wr):
    return pl.pallas_call(
        _proj_body,
        grid=(_NB,),
        in_specs=[
            pl.BlockSpec((BN, H), lambda i: (i, 0)),
            pl.BlockSpec((R, H, H), lambda i: (0, 0, 0)),
        ],
        out_specs=pl.BlockSpec((R, BN, H), lambda i: (0, i, 0)),
        out_shape=jax.ShapeDtypeStruct((R, N, H), jnp.float32),
    )(x, Wr)


def _sc_segsum(proj_flat, gidx_flat, dst_flat, zeros):
    mesh = plsc.VectorSubcoreMesh(core_axis_name="c", subcore_axis_name="s")

    @functools.partial(
        pl.kernel,
        out_type=jax.ShapeDtypeStruct((2, _NPAD, H), jnp.float32),
        mesh=mesh,
        scratch_types=[
            [pltpu.VMEM((2, _SC_CH), jnp.int32) for _ in range(_NSLOT)],
            [pltpu.VMEM((_SC_CH, H), jnp.float32) for _ in range(_NBUF)],
            pltpu.VMEM_SHARED((_NPAD, H), jnp.float32),
            [pltpu.SemaphoreType.DMA for _ in range(_NSLOT)],
            [pltpu.SemaphoreType.DMA for _ in range(_NBUF)],
            [pltpu.SemaphoreType.DMA for _ in range(_NBUF)],
        ],
    )
    def k(proj_hbm, gidx_hbm, dst_hbm, zeros_hbm, out_hbm,
          slots, bufs, agg_sh, sis, gsem, ssem):
        c = lax.axis_index("c")
        s = lax.axis_index("s")
        base = jnp.where(c == 0, s * (_SC_T0 * _SC_CH),
                         16 * _SC_T0 * _SC_CH + s * (_SC_T1 * _SC_CH))
        zr = _NPAD // 16
        pltpu.sync_copy(zeros_hbm.at[pl.ds(s * zr, zr)],
                        agg_sh.at[pl.ds(s * zr, zr)])

        def fetch(m, j):
            off = base + m * _SC_CH
            pltpu.async_copy(gidx_hbm.at[pl.ds(off, _SC_CH)],
                             slots[j].at[0], sis[j])
            pltpu.async_copy(dst_hbm.at[pl.ds(off, _SC_CH)],
                             slots[j].at[1], sis[j])

        def fetch_wait(j):
            for r in range(2):
                pltpu.make_async_copy(gidx_hbm.at[pl.ds(0, _SC_CH)],
                                      slots[j].at[r], sis[j]).wait()

        for j in range(3):
            fetch(j, j)
        for j in range(2):
            fetch_wait(j)
            pltpu.async_copy(proj_hbm.at[slots[j].at[0]], bufs[j], gsem[j])

        plsc.subcore_barrier()

        def chunk_work(g, j, T):
            m = g + j
            b = j % _NBUF
            sl = slots[j % _NSLOT]
            pltpu.make_async_copy(proj_hbm.at[sl.at[0]], bufs[b], gsem[b]).wait()
            pltpu.async_copy(bufs[b], agg_sh.at[sl.at[1]], ssem[b], add=True)

            jf = (j + 3) % _NSLOT

            @pl.when(m + 3 < T)
            def _():
                fetch(m + 3, jf)

            @pl.when(m >= 1)
            def _():
                b1 = (j + _NBUF - 1) % _NBUF
                pltpu.make_async_copy(bufs[b1], agg_sh.at[sl.at[1]],
                                      ssem[b1]).wait()

            jn = (j + 2) % _NSLOT

            @pl.when(m + 2 < T)
            def _():
                b2 = (j + 2) % _NBUF
                fetch_wait(jn)
                pltpu.async_copy(proj_hbm.at[slots[jn].at[0]], bufs[b2],
                                 gsem[b2])

        def run(T):
            @pl.loop(0, T, step=_NSLOT)
            def _(g):
                for j in range(_NSLOT):
                    chunk_work(g, j, T)
            b_last = (T - 1) % _NBUF
            pltpu.make_async_copy(bufs[b_last], agg_sh.at[slots[0].at[1]],
                                  ssem[b_last]).wait()

        @pl.when(c == 0)
        def _():
            run(_SC_T0)

        @pl.when(c == 1)
        def _():
            run(_SC_T1)

        plsc.subcore_barrier()
        pltpu.sync_copy(agg_sh.at[pl.ds(s * zr, zr)],
                        out_hbm.at[c].at[pl.ds(s * zr, zr)])

    return k(proj_flat, gidx_flat, dst_flat, zeros)


def _post_body(p, x, wl, bc, gp, bp, gf, bf, w1, b1, w2, b2, out):
    xv = x[...]
    h = p[0] + p[1] + _dot(xv, wl[...]) + bc[...]
    h = _gelu(_ln(h, gp[...], bp[...]))
    x1 = xv + h
    y = _ln(x1, gf[...], bf[...])
    y = _dot(_gelu(_dot(y, w1[...]) + b1[...]), w2[...]) + b2[...]
    out[...] = x1 + y


def _tc_post(part, x, wl, bc, gp, bp, gf, bf, w1, b1, w2, b2):
    return pl.pallas_call(
        _post_body,
        grid=(_NB,),
        in_specs=[
            pl.BlockSpec((2, BN, H), lambda i: (0, i, 0)),
            pl.BlockSpec((BN, H), lambda i: (i, 0)),
            _full((H, H)), _full((1, H)),
            _full((1, H)), _full((1, H)), _full((1, H)), _full((1, H)),
            _full((H, FF)), _full((1, FF)), _full((FF, H)), _full((1, H)),
        ],
        out_specs=pl.BlockSpec((BN, H), lambda i: (i, 0)),
        out_shape=jax.ShapeDtypeStruct((N, H), jnp.float32),
    )(part, x, wl, bc, gp, bp, gf, bf, w1, b1, w2, b2)


def _heads_body(x, gk, bk, wk1, bk1, wk2, bk2, gp, bp, wp1, bp1, wp2, bp2,
                keep, pri):
    xv = x[...]
    hk = _gelu(_dot(_ln(xv, gk[...], bk[...]), wk1[...]) + bk1[...])
    keep[...] = jnp.sum(hk * wk2[...], axis=1, keepdims=True) + bk2[...]
    hp = _gelu(_dot(_ln(xv, gp[...], bp[...]), wp1[...]) + bp1[...])
    pri[...] = jnp.sum(hp * wp2[...], axis=1, keepdims=True) + bp2[...]


def _tc_heads(x, gk, bk, wk1, bk1, wk2, bk2, gp, bp, wp1, bp1, wp2, bp2):
    return pl.pallas_call(
        _heads_body,
        grid=(_NB,),
        in_specs=[
            pl.BlockSpec((BN, H), lambda i: (i, 0)),
            _full((1, H)), _full((1, H)), _full((H, H)), _full((1, H)),
            _full((1, H)), _full((1, 1)),
            _full((1, H)), _full((1, H)), _full((H, H)), _full((1, H)),
            _full((1, H)), _full((1, 1)),
        ],
        out_specs=[
            pl.BlockSpec((BN, 1), lambda i: (i, 0)),
            pl.BlockSpec((BN, 1), lambda i: (i, 0)),
        ],
        out_shape=[
            jax.ShapeDtypeStruct((N, 1), jnp.float32),
            jax.ShapeDtypeStruct((N, 1), jnp.float32),
        ],
    )(x, gk, bk, wk1, bk1, wk2, bk2, gp, bp, wp1, bp1, wp2, bp2)


def _row(v):
    return v.reshape(1, -1)


def kernel(scalar_feat, type_feat, hyp_feat, edge_index, rel_type, ln_sc_g, ln_sc_b, W_sc, b_sc, W_type, W_hyp, ln_fu_g, ln_fu_b, W_fu, b_fu, Wrel, Wloop, b_conv, ln_pre_g, ln_pre_b, ln_ff_g, ln_ff_b, W_ff1, b_ff1, W_ff2, b_ff2, ln_k_g, ln_k_b, Wk1, bk1, Wk2, bk2, ln_p_g, ln_p_b, Wp1, bp1, Wp2, bp2):
    src = edge_index[0]
    dst = edge_index[1]
    gidx = rel_type * N + src
    pad = _E_PAD - E
    gidx_p = jnp.concatenate([gidx, jnp.zeros((pad,), jnp.int32)])
    dst_p = jnp.concatenate([dst, jnp.full((pad,), N, jnp.int32)])
    zeros = jnp.zeros((_NPAD, H), jnp.float32)

    Wcat = jnp.zeros((112, 64), jnp.float32)
    Wcat = Wcat.at[:16, :16].set(W_sc)
    Wcat = Wcat.at[16:80, 16:48].set(W_type)
    Wcat = Wcat.at[80:, 48:].set(W_hyp)
    bcat = jnp.concatenate([b_sc, jnp.zeros((48,), jnp.float32)])

    x = _tc_frontend(scalar_feat, type_feat, hyp_feat,
                     _row(ln_sc_g), _row(ln_sc_b), Wcat, _row(bcat),
                     _row(ln_fu_g), _row(ln_fu_b), W_fu, _row(b_fu))

    for i in range(L):
        proj = _tc_proj(x, Wrel[i])
        part = _sc_segsum(proj.reshape(R * N, H), gidx_p, dst_p, zeros)
        x = _tc_post(part, x, Wloop[i], _row(b_conv[i]),
                     _row(ln_pre_g[i]), _row(ln_pre_b[i]),
                     _row(ln_ff_g[i]), _row(ln_ff_b[i]),
                     W_ff1[i], _row(b_ff1[i]), W_ff2[i], _row(b_ff2[i]))

    keep, pri = _tc_heads(x, _row(ln_k_g), _row(ln_k_b), Wk1, _row(bk1),
                          _row(Wk2), bk2.reshape(1, 1),
                          _row(ln_p_g), _row(ln_p_b),
                          Wp1, _row(bp1), _row(Wp2), bp2.reshape(1, 1))
    return jnp.stack([keep[:, 0], pri[:, 0]], axis=0)
